# baseline (speedup 1.0000x reference)
import dataclasses
import functools

import jax
import jax.numpy as jnp
from jax import lax
from jax.experimental import pallas as pl
from jax.experimental.pallas import tpu as pltpu
from jax.experimental.pallas import tpu_sc as plsc

BATCH = 16384
EMBED = 64
NUM_CORES = 2
NUM_SUBCORES = 16
LANES = 16
NUM_WORKERS = NUM_CORES * NUM_SUBCORES
CHUNK = BATCH // NUM_WORKERS
GROUPS = CHUNK // LANES


def _bpr_body(batch_hbm, user_hbm, item_hbm, out_hbm,
              slab, uidx, pidx, nidx, urows, prows, nrows, outv, sem):
    wid = lax.axis_index("s") * NUM_CORES + lax.axis_index("c")
    base = wid * CHUNK

    pltpu.sync_copy(batch_hbm.at[pl.ds(base, CHUNK)], slab)

    lanes = lax.iota(jnp.int32, LANES)

    @pl.loop(0, GROUPS)
    def _extract(g):
        rows = g * LANES + lanes
        u = plsc.load_gather(slab, [rows, jnp.zeros((LANES,), jnp.int32)])
        p = plsc.load_gather(slab, [rows, jnp.ones((LANES,), jnp.int32)])
        n = plsc.load_gather(slab, [rows, jnp.full((LANES,), 2, jnp.int32)])
        uidx[pl.ds(g * LANES, LANES)] = u
        pidx[pl.ds(g * LANES, LANES)] = p
        nidx[pl.ds(g * LANES, LANES)] = n

    cp_u = pltpu.async_copy(user_hbm.at[uidx], urows, sem)
    cp_p = pltpu.async_copy(item_hbm.at[pidx], prows, sem)
    cp_n = pltpu.async_copy(item_hbm.at[nidx], nrows, sem)
    cp_u.wait()
    cp_p.wait()
    cp_n.wait()

    @pl.loop(0, GROUPS)
    def _dot(g):
        rows = g * LANES + lanes
        acc = jnp.zeros((LANES,), jnp.float32)
        for d in range(EMBED):
            cols = (lanes + d) & (EMBED - 1)
            pv = plsc.load_gather(prows, [rows, cols])
            nv = plsc.load_gather(nrows, [rows, cols])
            uv = plsc.load_gather(urows, [rows, cols])
            acc = acc + uv * (pv - nv)
        outv[pl.ds(g * LANES, LANES)] = acc

    pltpu.sync_copy(outv, out_hbm.at[pl.ds(base, CHUNK)])


@jax.jit
def _bpr_sc(batch, user_memory, item_memory):
    mesh = plsc.VectorSubcoreMesh(core_axis_name="c", subcore_axis_name="s")
    cp = pltpu.CompilerParams(
        needs_layout_passes=False,
        use_tc_tiling_on_sc=False,
    )
    run = pl.kernel(
        _bpr_body,
        out_type=jax.ShapeDtypeStruct((BATCH,), jnp.float32),
        mesh=mesh,
        scratch_types=[
            pltpu.VMEM((CHUNK, 3), jnp.int32),
            pltpu.VMEM((CHUNK,), jnp.int32),
            pltpu.VMEM((CHUNK,), jnp.int32),
            pltpu.VMEM((CHUNK,), jnp.int32),
            pltpu.VMEM((CHUNK, EMBED), jnp.float32),
            pltpu.VMEM((CHUNK, EMBED), jnp.float32),
            pltpu.VMEM((CHUNK, EMBED), jnp.float32),
            pltpu.VMEM((CHUNK,), jnp.float32),
            pltpu.SemaphoreType.DMA,
        ],
        compiler_params=cp,
    )
    return run(batch, user_memory, item_memory)


def kernel(batch, user_memory, item_memory):
    return _bpr_sc(batch, user_memory, item_memory)
